# baseline (speedup 1.0000x reference)
import jax
import jax.numpy as jnp
from jax.experimental import pallas as pl
from jax.experimental.pallas import tpu as pltpu

_EPS = 1e-06
_LOG2E = 1.4426950408889634
_BK_BLOCK = 8
_N_MXU = 4


def _vpu_tile(slab):
    slab_t = slab.T
    wneg = (-0.5 * _LOG2E) / (_EPS + slab[8:12, :])
    acc = None
    for r in range(4):
        xc = slab_t[:, r:r + 1]
        mr = slab[4 + r:5 + r, :]
        wr = wneg[r:r + 1, :]
        d = xc - mr
        term = d * d * wr
        acc = term if acc is None else acc + term
    return jax.lax.exp2(acc)


def _mxu_tile(slab):
    x = slab[0:4, :]
    m = slab[4:8, :]
    w = 1.0 / (_EPS + slab[8:12, :])
    wm = w * m
    a_rows = jnp.concatenate(
        [x * x, x, jnp.full((1, 256), 1.0, jnp.float32)], axis=0
    )
    b_mat = jnp.concatenate(
        [
            w * (-0.5 * _LOG2E),
            wm * _LOG2E,
            jnp.sum(wm * m, axis=0, keepdims=True) * (-0.5 * _LOG2E),
        ],
        axis=0,
    )
    prod = jnp.dot(a_rows.T, b_mat, preferred_element_type=jnp.float32,
                   precision=jax.lax.Precision.HIGHEST)
    return jax.lax.exp2(prod)


def _densities_kernel(slab_ref, out_ref):
    nblk = slab_ref.shape[0]
    for j in range(nblk):
        slab = slab_ref[j]
        if j < _N_MXU:
            out_ref[j] = _mxu_tile(slab)
        else:
            out_ref[j] = _vpu_tile(slab)


def kernel(points, means, sigmas):
    b, k, l, rank = points.shape
    c = means.shape[2]
    bk = b * k
    stacked = jnp.concatenate(
        [
            points.reshape(bk, l, rank),
            means.reshape(bk, c, rank),
            sigmas.reshape(bk, c, rank),
        ],
        axis=2,
    )
    slab = stacked.transpose(0, 2, 1)

    out = pl.pallas_call(
        _densities_kernel,
        grid=(bk // _BK_BLOCK,),
        in_specs=[
            pl.BlockSpec((_BK_BLOCK, 12, 256), lambda i: (i, 0, 0)),
        ],
        out_specs=pl.BlockSpec((_BK_BLOCK, l, c), lambda i: (i, 0, 0)),
        out_shape=jax.ShapeDtypeStruct((bk, l, c), jnp.float32),
        compiler_params=pltpu.CompilerParams(
            dimension_semantics=("arbitrary",),
        ),
    )(slab)
    return out.reshape(b, k, l, c)


# speedup vs baseline: 1.0406x; 1.0406x over previous
import jax
import jax.numpy as jnp
from jax.experimental import pallas as pl
from jax.experimental.pallas import tpu as pltpu

_EPS = 1e-06
_LOG2E = 1.4426950408889634
_BK_BLOCK = 16
_N_MXU = 8


def _vpu_tile(slab):
    slab_t = slab.T
    wneg = (-0.5 * _LOG2E) / (_EPS + slab[8:12, :])
    acc = None
    for r in range(4):
        xc = slab_t[:, r:r + 1]
        mr = slab[4 + r:5 + r, :]
        wr = wneg[r:r + 1, :]
        d = xc - mr
        term = d * d * wr
        acc = term if acc is None else acc + term
    return jax.lax.exp2(acc)


def _mxu_tile(slab):
    x = slab[0:4, :]
    m = slab[4:8, :]
    w = 1.0 / (_EPS + slab[8:12, :])
    wm = w * m
    a_rows = jnp.concatenate(
        [x * x, x, jnp.full((1, 256), 1.0, jnp.float32)], axis=0
    )
    b_mat = jnp.concatenate(
        [
            w * (-0.5 * _LOG2E),
            wm * _LOG2E,
            jnp.sum(wm * m, axis=0, keepdims=True) * (-0.5 * _LOG2E),
        ],
        axis=0,
    )
    prod = jnp.dot(a_rows.T, b_mat, preferred_element_type=jnp.float32,
                   precision=jax.lax.Precision.HIGHEST)
    return jax.lax.exp2(prod)


def _densities_kernel(slab_ref, out_ref):
    nblk = slab_ref.shape[0]
    for j in range(nblk):
        slab = slab_ref[j]
        if j < _N_MXU:
            out_ref[j] = _mxu_tile(slab)
        else:
            out_ref[j] = _vpu_tile(slab)


def kernel(points, means, sigmas):
    b, k, l, rank = points.shape
    c = means.shape[2]
    bk = b * k
    stacked = jnp.concatenate(
        [
            points.reshape(bk, l, rank),
            means.reshape(bk, c, rank),
            sigmas.reshape(bk, c, rank),
        ],
        axis=2,
    )
    slab = stacked.transpose(0, 2, 1)

    out = pl.pallas_call(
        _densities_kernel,
        grid=(bk // _BK_BLOCK,),
        in_specs=[
            pl.BlockSpec((_BK_BLOCK, 12, 256), lambda i: (i, 0, 0)),
        ],
        out_specs=pl.BlockSpec((_BK_BLOCK, l, c), lambda i: (i, 0, 0)),
        out_shape=jax.ShapeDtypeStruct((bk, l, c), jnp.float32),
        compiler_params=pltpu.CompilerParams(
            dimension_semantics=("arbitrary",),
        ),
    )(slab)
    return out.reshape(b, k, l, c)
